# Initial kernel scaffold; baseline (speedup 1.0000x reference)
import jax
import jax.numpy as jnp
from jax.experimental import pallas as pl


def kernel(x, edge_index, W1, b1, g1, bt1, W2, b2, g2, bt2):
    raise NotImplementedError("write your pallas kernel here")


# speedup vs baseline: 15.7375x; 15.7375x over previous
import dataclasses
import functools

import jax
import jax.numpy as jnp
from jax import lax
from jax.experimental import pallas as pl
from jax.experimental.pallas import tpu as pltpu
from jax.experimental.pallas import tpu_sc as plsc

N = 10000
D = 128
E = 320000
EPS = 1e-5

NC = 2
NS = 16
NW = NC * NS

CH = 128
NCHUNK = 79
E_PW = NCHUNK * CH
E_PAD = E_PW * NW
ACC_ROWS = 10240
RPT = ACC_ROWS // NS
DEG_ROWS = 128

_mesh = plsc.VectorSubcoreMesh(core_axis_name="c", subcore_axis_name="s")

_sc_params = pltpu.CompilerParams()
if "needs_layout_passes" in pltpu.CompilerParams.__dataclass_fields__:
    _sc_params = dataclasses.replace(_sc_params, needs_layout_passes=False)


@functools.partial(
    pl.kernel,
    out_type=jax.ShapeDtypeStruct((NC, DEG_ROWS, 128), jnp.float32),
    mesh=_mesh,
    compiler_params=_sc_params,
    scratch_types=[
        pltpu.VMEM((E_PW,), jnp.int32),
        pltpu.VMEM((DEG_ROWS, 128), jnp.float32),
        pltpu.VMEM((DEG_ROWS,), jnp.int32),
        pltpu.VMEM((DEG_ROWS // NS, 128), jnp.float32),
        pltpu.VMEM_SHARED((DEG_ROWS, 128), jnp.float32),
    ],
)
def _deg_kernel(dst_hbm, out_hbm, dst_v, hist_v, rowidx_v, stage_v, acc_sh):
    c = lax.axis_index("c")
    s = lax.axis_index("s")
    wid = c * NS + s
    zeros16 = jnp.zeros((16,), jnp.float32)
    ones16 = jnp.ones((16,), jnp.float32)

    @pl.loop(0, DEG_ROWS)
    def _(r):
        @pl.loop(0, 8)
        def _(j):
            hist_v[r, pl.ds(j * 16, 16)] = zeros16

    @pl.loop(0, DEG_ROWS // 16)
    def _(j):
        rowidx_v[pl.ds(j * 16, 16)] = lax.iota(jnp.int32, 16) + j * 16

    pltpu.sync_copy(dst_hbm.at[pl.ds(wid * E_PW, E_PW)], dst_v)

    @pl.loop(0, E_PW // 16)
    def _(i):
        idx = dst_v[pl.ds(i * 16, 16)]
        q = lax.shift_right_logical(idx, 7)
        r = lax.bitwise_and(idx, 127)
        plsc.addupdate_scatter(hist_v, [q, r], ones16)

    @pl.when(s == 0)
    def _():
        pltpu.sync_copy(hist_v, acc_sh)

    plsc.subcore_barrier()

    @pl.when(s != 0)
    def _():
        pltpu.sync_copy(hist_v, acc_sh.at[rowidx_v], add=True)

    plsc.subcore_barrier()
    rpt = DEG_ROWS // NS
    pltpu.sync_copy(acc_sh.at[pl.ds(s * rpt, rpt), :], stage_v)
    pltpu.sync_copy(stage_v, out_hbm.at[c, pl.ds(s * rpt, rpt), :])


@functools.partial(
    pl.kernel,
    out_type=jax.ShapeDtypeStruct((NC, ACC_ROWS, D), jnp.float32),
    mesh=_mesh,
    compiler_params=_sc_params,
    scratch_types=[
        pltpu.VMEM((CH,), jnp.int32),
        pltpu.VMEM((CH,), jnp.int32),
        pltpu.VMEM((CH, D), jnp.float32),
        pltpu.VMEM_SHARED((ACC_ROWS, D), jnp.float32),
    ],
)
def _edge_scatter_kernel(table_hbm, src_hbm, dst_hbm, out_hbm,
                         src_v, dst_v, rows_v, acc_sh):
    c = lax.axis_index("c")
    s = lax.axis_index("s")
    wid = c * NS + s
    zeros16 = jnp.zeros((16,), jnp.float32)

    @pl.loop(0, CH)
    def _(r):
        @pl.loop(0, D // 16)
        def _(j):
            rows_v[r, pl.ds(j * 16, 16)] = zeros16

    @pl.loop(0, RPT // CH)
    def _(j):
        pltpu.sync_copy(rows_v, acc_sh.at[pl.ds(s * RPT + j * CH, CH), :])

    plsc.subcore_barrier()

    base = wid * E_PW

    @pl.loop(0, NCHUNK)
    def _(k):
        off = base + k * CH
        pltpu.sync_copy(src_hbm.at[pl.ds(off, CH)], src_v)
        pltpu.sync_copy(dst_hbm.at[pl.ds(off, CH)], dst_v)
        pltpu.sync_copy(table_hbm.at[src_v], rows_v)
        pltpu.sync_copy(rows_v, acc_sh.at[dst_v], add=True)

    plsc.subcore_barrier()

    @pl.loop(0, RPT // CH)
    def _(j):
        pltpu.sync_copy(acc_sh.at[pl.ds(s * RPT + j * CH, CH), :], rows_v)
        pltpu.sync_copy(rows_v, out_hbm.at[c, pl.ds(s * RPT + j * CH, CH), :])


def _mm_scale_body(x_ref, w_ref, d0_ref, d1_ref, hp_ref, dinv_ref):
    deg = d0_ref[...] + d1_ref[...] + 1.0
    dinv = lax.rsqrt(deg)
    dinv_ref[...] = dinv
    h = lax.dot_general(x_ref[...], w_ref[...], (((1,), (1,)), ((), ())),
                        preferred_element_type=jnp.float32)
    hp_ref[...] = h * dinv


def _mid_body(a0_ref, a1_ref, hp_ref, dinv_ref, b_ref, g_ref, bt_ref, w_ref,
              out_ref):
    dv = dinv_ref[...]
    y = dv * (a0_ref[...] + a1_ref[...] + hp_ref[...]) + b_ref[...]
    mean = jnp.sum(y, axis=0, keepdims=True) * (1.0 / N)
    var = jnp.sum(y * y, axis=0, keepdims=True) * (1.0 / N) - mean * mean
    z = g_ref[...] * (y - mean) * lax.rsqrt(var + EPS) + bt_ref[...]
    z = jnp.maximum(z, 0.0)
    h2 = lax.dot_general(z, w_ref[...], (((1,), (1,)), ((), ())),
                         preferred_element_type=jnp.float32)
    out_ref[...] = h2 * dv


def _final_body(a0_ref, a1_ref, hp_ref, dinv_ref, b_ref, g_ref, bt_ref, x_ref,
                out_ref):
    dv = dinv_ref[...]
    y = dv * (a0_ref[...] + a1_ref[...] + hp_ref[...]) + b_ref[...]
    mean = jnp.sum(y, axis=0, keepdims=True) * (1.0 / N)
    var = jnp.sum(y * y, axis=0, keepdims=True) * (1.0 / N) - mean * mean
    z = g_ref[...] * (y - mean) * lax.rsqrt(var + EPS) + bt_ref[...]
    out_ref[...] = jnp.maximum(z + x_ref[...], 0.0)


def _f32(*shapes):
    return [jax.ShapeDtypeStruct(s, jnp.float32) for s in shapes]


def kernel(x, edge_index, W1, b1, g1, bt1, W2, b2, g2, bt2):
    src = edge_index[0].astype(jnp.int32)
    dst = edge_index[1].astype(jnp.int32)
    npad = E_PAD - E
    pad_i = jnp.arange(npad, dtype=jnp.int32)
    src_p = jnp.concatenate([src, pad_i % N])
    dst_p = jnp.concatenate([dst, N + pad_i % (ACC_ROWS - N)])

    degp = _deg_kernel(dst_p)
    degf = degp.reshape(NC, DEG_ROWS * 128)[:, :N]
    d0 = degf[0][:, None]
    d1 = degf[1][:, None]

    h1p, dinv = pl.pallas_call(
        _mm_scale_body,
        out_shape=_f32((N, D), (N, 1)),
    )(x, W1, d0, d1)

    acc1 = _edge_scatter_kernel(h1p, src_p, dst_p)

    h2p = pl.pallas_call(
        _mid_body,
        out_shape=_f32((N, D))[0],
    )(acc1[0, :N, :], acc1[1, :N, :], h1p, dinv,
      b1[None, :], g1[None, :], bt1[None, :], W2)

    acc2 = _edge_scatter_kernel(h2p, src_p, dst_p)

    out = pl.pallas_call(
        _final_body,
        out_shape=_f32((N, D))[0],
    )(acc2[0, :N, :], acc2[1, :N, :], h2p, dinv,
      b2[None, :], g2[None, :], bt2[None, :], x)
    return out


# speedup vs baseline: 27.2575x; 1.7320x over previous
import dataclasses
import functools

import jax
import jax.numpy as jnp
from jax import lax
from jax.experimental import pallas as pl
from jax.experimental.pallas import tpu as pltpu
from jax.experimental.pallas import tpu_sc as plsc

N = 10000
D = 128
E = 320000
EPS = 1e-5

NC = 2
NS = 16
NW = NC * NS

CH = 128
NCHUNK = 80
E_PW = NCHUNK * CH
E_PAD = E_PW * NW
ACC_ROWS = 10240
RPT = ACC_ROWS // NS
DEG_ROWS = 128

_mesh = plsc.VectorSubcoreMesh(core_axis_name="c", subcore_axis_name="s")

_sc_params = pltpu.CompilerParams()
if "needs_layout_passes" in pltpu.CompilerParams.__dataclass_fields__:
    _sc_params = dataclasses.replace(_sc_params, needs_layout_passes=False)


@functools.partial(
    pl.kernel,
    out_type=jax.ShapeDtypeStruct((NC, DEG_ROWS, 128), jnp.float32),
    mesh=_mesh,
    compiler_params=_sc_params,
    scratch_types=[
        pltpu.VMEM((NCHUNK, CH), jnp.int32),
        pltpu.VMEM((DEG_ROWS, 128), jnp.float32),
        pltpu.VMEM((DEG_ROWS,), jnp.int32),
        pltpu.VMEM((DEG_ROWS // NS, 128), jnp.float32),
        pltpu.VMEM_SHARED((DEG_ROWS, 128), jnp.float32),
        pltpu.SemaphoreType.DMA,
    ],
)
def _deg_kernel(dst_hbm, out_hbm, dst_v, hist_v, rowidx_v, stage_v, acc_sh,
                sem):
    c = lax.axis_index("c")
    s = lax.axis_index("s")
    wid = c * NS + s
    zeros16 = jnp.zeros((16,), jnp.float32)
    ones16 = jnp.ones((16,), jnp.float32)

    cp = pltpu.async_copy(dst_hbm.at[wid], dst_v, sem)

    @pl.loop(0, DEG_ROWS)
    def _(r):
        @pl.loop(0, 8)
        def _(j):
            hist_v[r, pl.ds(j * 16, 16)] = zeros16

    @pl.loop(0, DEG_ROWS // 16)
    def _(j):
        rowidx_v[pl.ds(j * 16, 16)] = lax.iota(jnp.int32, 16) + j * 16

    cp.wait()

    @pl.loop(0, E_PW // 16)
    def _(i):
        idx = dst_v[i >> 3, pl.ds((i & 7) * 16, 16)]
        q = lax.shift_right_logical(idx, 7)
        r = lax.bitwise_and(idx, 127)
        plsc.addupdate_scatter(hist_v, [q, r], ones16)

    @pl.when(s == 0)
    def _():
        pltpu.sync_copy(hist_v, acc_sh)

    plsc.subcore_barrier()

    @pl.when(s != 0)
    def _():
        pltpu.sync_copy(hist_v, acc_sh.at[rowidx_v], add=True)

    plsc.subcore_barrier()
    rpt = DEG_ROWS // NS
    pltpu.sync_copy(acc_sh.at[pl.ds(s * rpt, rpt), :], stage_v)
    pltpu.sync_copy(stage_v, out_hbm.at[c, pl.ds(s * rpt, rpt), :])


@functools.partial(
    pl.kernel,
    out_type=jax.ShapeDtypeStruct((NC, ACC_ROWS, D), jnp.float32),
    mesh=_mesh,
    compiler_params=_sc_params,
    scratch_types=[
        pltpu.VMEM((2, CH), jnp.int32),
        pltpu.VMEM((2, CH), jnp.int32),
        pltpu.VMEM((CH, D), jnp.float32),
        pltpu.VMEM((CH, D), jnp.float32),
        pltpu.VMEM_SHARED((ACC_ROWS, D), jnp.float32),
        pltpu.SemaphoreType.DMA,
        pltpu.SemaphoreType.DMA,
        pltpu.SemaphoreType.DMA,
        pltpu.SemaphoreType.DMA,
    ],
)
def _edge_scatter_kernel(table_hbm, src_hbm, dst_hbm, out_hbm,
                         sidx_v, didx_v, rows0_v, rows1_v, acc_sh,
                         semg0, semg1, semi0, semi1):
    c = lax.axis_index("c")
    s = lax.axis_index("s")
    wid = c * NS + s
    zeros16 = jnp.zeros((16,), jnp.float32)

    cps0 = pltpu.async_copy(src_hbm.at[wid, 0], sidx_v.at[0], semi0)
    cpd0 = pltpu.async_copy(dst_hbm.at[wid, 0], didx_v.at[0], semi0)
    cps1 = pltpu.async_copy(src_hbm.at[wid, 1], sidx_v.at[1], semi1)
    cpd1 = pltpu.async_copy(dst_hbm.at[wid, 1], didx_v.at[1], semi1)

    @pl.loop(0, CH)
    def _(r):
        @pl.loop(0, D // 16)
        def _(j):
            rows0_v[r, pl.ds(j * 16, 16)] = zeros16

    @pl.loop(0, RPT // CH)
    def _(j):
        pltpu.sync_copy(rows0_v, acc_sh.at[pl.ds(s * RPT + j * CH, CH), :])

    plsc.subcore_barrier()

    cps0.wait()
    cpd0.wait()
    pltpu.async_copy(table_hbm.at[sidx_v.at[0]], rows0_v, semg0)
    cps1.wait()
    cpd1.wait()
    pltpu.async_copy(table_hbm.at[sidx_v.at[1]], rows1_v, semg1)

    @pl.loop(0, NCHUNK // 2)
    def _(t):
        k0 = 2 * t
        pltpu.make_async_copy(table_hbm.at[sidx_v.at[0]], rows0_v,
                              semg0).wait()
        pltpu.sync_copy(rows0_v, acc_sh.at[didx_v.at[0]], add=True)

        @pl.when(k0 + 2 < NCHUNK)
        def _():
            pltpu.async_copy(src_hbm.at[wid, k0 + 2], sidx_v.at[0], semi0)
            pltpu.async_copy(dst_hbm.at[wid, k0 + 2], didx_v.at[0], semi0)
            pltpu.make_async_copy(src_hbm.at[wid, k0 + 2], sidx_v.at[0],
                                  semi0).wait()
            pltpu.make_async_copy(dst_hbm.at[wid, k0 + 2], didx_v.at[0],
                                  semi0).wait()
            pltpu.async_copy(table_hbm.at[sidx_v.at[0]], rows0_v, semg0)

        pltpu.make_async_copy(table_hbm.at[sidx_v.at[1]], rows1_v,
                              semg1).wait()
        pltpu.sync_copy(rows1_v, acc_sh.at[didx_v.at[1]], add=True)

        @pl.when(k0 + 3 < NCHUNK)
        def _():
            pltpu.async_copy(src_hbm.at[wid, k0 + 3], sidx_v.at[1], semi1)
            pltpu.async_copy(dst_hbm.at[wid, k0 + 3], didx_v.at[1], semi1)
            pltpu.make_async_copy(src_hbm.at[wid, k0 + 3], sidx_v.at[1],
                                  semi1).wait()
            pltpu.make_async_copy(dst_hbm.at[wid, k0 + 3], didx_v.at[1],
                                  semi1).wait()
            pltpu.async_copy(table_hbm.at[sidx_v.at[1]], rows1_v, semg1)

    plsc.subcore_barrier()

    @pl.loop(0, RPT // CH)
    def _(j):
        pltpu.sync_copy(acc_sh.at[pl.ds(s * RPT + j * CH, CH), :], rows0_v)
        pltpu.sync_copy(rows0_v, out_hbm.at[c, pl.ds(s * RPT + j * CH, CH), :])


def _mm_scale_body(x_ref, w_ref, d0_ref, d1_ref, hp_ref, dinv_ref):
    deg = d0_ref[...] + d1_ref[...] + 1.0
    dinv = lax.rsqrt(deg)
    dinv_ref[...] = dinv
    h = lax.dot_general(x_ref[...], w_ref[...], (((1,), (1,)), ((), ())),
                        preferred_element_type=jnp.float32)
    hp_ref[...] = h * dinv


def _mid_body(a0_ref, a1_ref, hp_ref, dinv_ref, b_ref, g_ref, bt_ref, w_ref,
              out_ref):
    dv = dinv_ref[...]
    y = dv * (a0_ref[...] + a1_ref[...] + hp_ref[...]) + b_ref[...]
    mean = jnp.sum(y, axis=0, keepdims=True) * (1.0 / N)
    var = jnp.sum(y * y, axis=0, keepdims=True) * (1.0 / N) - mean * mean
    z = g_ref[...] * (y - mean) * lax.rsqrt(var + EPS) + bt_ref[...]
    z = jnp.maximum(z, 0.0)
    h2 = lax.dot_general(z, w_ref[...], (((1,), (1,)), ((), ())),
                         preferred_element_type=jnp.float32)
    out_ref[...] = h2 * dv


def _final_body(a0_ref, a1_ref, hp_ref, dinv_ref, b_ref, g_ref, bt_ref, x_ref,
                out_ref):
    dv = dinv_ref[...]
    y = dv * (a0_ref[...] + a1_ref[...] + hp_ref[...]) + b_ref[...]
    mean = jnp.sum(y, axis=0, keepdims=True) * (1.0 / N)
    var = jnp.sum(y * y, axis=0, keepdims=True) * (1.0 / N) - mean * mean
    z = g_ref[...] * (y - mean) * lax.rsqrt(var + EPS) + bt_ref[...]
    out_ref[...] = jnp.maximum(z + x_ref[...], 0.0)


def _f32(*shapes):
    return [jax.ShapeDtypeStruct(s, jnp.float32) for s in shapes]


def kernel(x, edge_index, W1, b1, g1, bt1, W2, b2, g2, bt2):
    src = edge_index[0].astype(jnp.int32)
    dst = edge_index[1].astype(jnp.int32)
    npad = E_PAD - E
    pad_i = jnp.arange(npad, dtype=jnp.int32)
    src_p = jnp.concatenate([src, pad_i % N]).reshape(NW, NCHUNK, CH)
    dst_p = jnp.concatenate([dst, N + pad_i % (ACC_ROWS - N)]
                            ).reshape(NW, NCHUNK, CH)

    degp = _deg_kernel(dst_p)
    degf = degp.reshape(NC, DEG_ROWS * 128)[:, :N]
    d0 = degf[0][:, None]
    d1 = degf[1][:, None]

    h1p, dinv = pl.pallas_call(
        _mm_scale_body,
        out_shape=_f32((N, D), (N, 1)),
    )(x, W1, d0, d1)

    acc1 = _edge_scatter_kernel(h1p, src_p, dst_p)

    h2p = pl.pallas_call(
        _mid_body,
        out_shape=_f32((N, D))[0],
    )(acc1[0, :N, :], acc1[1, :N, :], h1p, dinv,
      b1[None, :], g1[None, :], bt1[None, :], W2)

    acc2 = _edge_scatter_kernel(h2p, src_p, dst_p)

    out = pl.pallas_call(
        _final_body,
        out_shape=_f32((N, D))[0],
    )(acc2[0, :N, :], acc2[1, :N, :], h2p, dinv,
      b2[None, :], g2[None, :], bt2[None, :], x)
    return out


# speedup vs baseline: 29.9918x; 1.1003x over previous
import dataclasses
import functools

import jax
import jax.numpy as jnp
from jax import lax
from jax.experimental import pallas as pl
from jax.experimental.pallas import tpu as pltpu
from jax.experimental.pallas import tpu_sc as plsc

N = 10000
D = 128
E = 320000
EPS = 1e-5

NC = 2
NS = 16
NW = NC * NS

CH = 128
NCHUNK = 80
E_PW = NCHUNK * CH
E_PAD = E_PW * NW
ACC_ROWS = 10240
RPT = ACC_ROWS // NS
DEG_ROWS = 128

_mesh = plsc.VectorSubcoreMesh(core_axis_name="c", subcore_axis_name="s")

_sc_params = pltpu.CompilerParams()
if "needs_layout_passes" in pltpu.CompilerParams.__dataclass_fields__:
    _sc_params = dataclasses.replace(_sc_params, needs_layout_passes=False)


@functools.partial(
    pl.kernel,
    out_type=jax.ShapeDtypeStruct((NC, DEG_ROWS, 128), jnp.float32),
    mesh=_mesh,
    compiler_params=_sc_params,
    scratch_types=[
        pltpu.VMEM((NCHUNK, CH), jnp.int32),
        pltpu.VMEM((DEG_ROWS, 128), jnp.float32),
        pltpu.VMEM((DEG_ROWS,), jnp.int32),
        pltpu.VMEM((DEG_ROWS // NS, 128), jnp.float32),
        pltpu.VMEM_SHARED((DEG_ROWS, 128), jnp.float32),
        pltpu.SemaphoreType.DMA,
    ],
)
def _deg_kernel(dst_hbm, out_hbm, dst_v, hist_v, rowidx_v, stage_v, acc_sh,
                sem):
    c = lax.axis_index("c")
    s = lax.axis_index("s")
    wid = c * NS + s
    zeros16 = jnp.zeros((16,), jnp.float32)
    ones16 = jnp.ones((16,), jnp.float32)

    cp = pltpu.async_copy(dst_hbm.at[wid], dst_v, sem)

    @pl.loop(0, DEG_ROWS)
    def _(r):
        @pl.loop(0, 8)
        def _(j):
            hist_v[r, pl.ds(j * 16, 16)] = zeros16

    @pl.loop(0, DEG_ROWS // 16)
    def _(j):
        rowidx_v[pl.ds(j * 16, 16)] = lax.iota(jnp.int32, 16) + j * 16

    cp.wait()

    @pl.loop(0, E_PW // 16)
    def _(i):
        idx = dst_v[i >> 3, pl.ds((i & 7) * 16, 16)]
        q = lax.shift_right_logical(idx, 7)
        r = lax.bitwise_and(idx, 127)
        plsc.addupdate_scatter(hist_v, [q, r], ones16)

    @pl.when(s == 0)
    def _():
        pltpu.sync_copy(hist_v, acc_sh)

    plsc.subcore_barrier()

    @pl.when(s != 0)
    def _():
        pltpu.sync_copy(hist_v, acc_sh.at[rowidx_v], add=True)

    plsc.subcore_barrier()
    rpt = DEG_ROWS // NS
    pltpu.sync_copy(acc_sh.at[pl.ds(s * rpt, rpt), :], stage_v)
    pltpu.sync_copy(stage_v, out_hbm.at[c, pl.ds(s * rpt, rpt), :])


@functools.partial(
    pl.kernel,
    out_type=jax.ShapeDtypeStruct((NC, ACC_ROWS, D), jnp.float32),
    mesh=_mesh,
    compiler_params=_sc_params,
    scratch_types=[
        pltpu.VMEM((4, CH), jnp.int32),
        pltpu.VMEM((4, CH), jnp.int32),
        pltpu.VMEM((CH, D), jnp.float32),
        pltpu.VMEM((CH, D), jnp.float32),
        pltpu.VMEM_SHARED((ACC_ROWS, D), jnp.float32),
        pltpu.SemaphoreType.DMA,
        pltpu.SemaphoreType.DMA,
        pltpu.SemaphoreType.DMA,
        pltpu.SemaphoreType.DMA,
        pltpu.SemaphoreType.DMA,
        pltpu.SemaphoreType.DMA,
    ],
)
def _edge_scatter_kernel(table_hbm, src_hbm, dst_hbm, out_hbm,
                         sidx_v, didx_v, rows0_v, rows1_v, acc_sh,
                         semg0, semg1, semi0, semi1, semi2, semi3):
    c = lax.axis_index("c")
    s = lax.axis_index("s")
    wid = c * NS + s
    zeros16 = jnp.zeros((16,), jnp.float32)
    semg = (semg0, semg1)
    semi = (semi0, semi1, semi2, semi3)
    rows = (rows0_v, rows1_v)

    def idx_load(k, q):
        pltpu.async_copy(src_hbm.at[wid, k], sidx_v.at[q], semi[q])
        pltpu.async_copy(dst_hbm.at[wid, k], didx_v.at[q], semi[q])

    def idx_wait(k, q):
        pltpu.make_async_copy(src_hbm.at[wid, k], sidx_v.at[q],
                              semi[q]).wait()
        pltpu.make_async_copy(dst_hbm.at[wid, k], didx_v.at[q],
                              semi[q]).wait()

    def gather_start(q, p):
        pltpu.async_copy(table_hbm.at[sidx_v.at[q]], rows[p], semg[p])

    def gather_wait(q, p):
        pltpu.make_async_copy(table_hbm.at[sidx_v.at[q]], rows[p],
                              semg[p]).wait()

    for q in range(4):
        idx_load(q, q)

    @pl.loop(0, CH)
    def _(r):
        @pl.loop(0, D // 16)
        def _(j):
            rows0_v[r, pl.ds(j * 16, 16)] = zeros16

    @pl.loop(0, RPT // CH)
    def _(j):
        pltpu.sync_copy(rows0_v, acc_sh.at[pl.ds(s * RPT + j * CH, CH), :])

    plsc.subcore_barrier()

    idx_wait(0, 0)
    gather_start(0, 0)
    idx_wait(1, 1)
    gather_start(1, 1)

    @pl.loop(0, NCHUNK // 4)
    def _(t):
        k0 = 4 * t
        for u in range(4):
            q = u
            p = u % 2
            k = k0 + u
            gather_wait(q, p)
            pltpu.sync_copy(rows[p], acc_sh.at[didx_v.at[q]], add=True)

            @pl.when(k + 4 < NCHUNK)
            def _():
                idx_load(k + 4, q)

            @pl.when(k + 2 < NCHUNK)
            def _():
                idx_wait(k + 2, (q + 2) % 4)
                gather_start((q + 2) % 4, p)

    plsc.subcore_barrier()

    @pl.loop(0, RPT // CH)
    def _(j):
        pltpu.sync_copy(acc_sh.at[pl.ds(s * RPT + j * CH, CH), :], rows0_v)
        pltpu.sync_copy(rows0_v, out_hbm.at[c, pl.ds(s * RPT + j * CH, CH), :])


def _mm_scale_body(x_ref, w_ref, d0_ref, d1_ref, hp_ref, dinv_ref):
    deg = d0_ref[...] + d1_ref[...] + 1.0
    dinv = lax.rsqrt(deg)
    dinv_ref[...] = dinv
    h = lax.dot_general(x_ref[...], w_ref[...], (((1,), (1,)), ((), ())),
                        preferred_element_type=jnp.float32)
    hp_ref[...] = h * dinv


def _mid_body(a0_ref, a1_ref, hp_ref, dinv_ref, b_ref, g_ref, bt_ref, w_ref,
              out_ref):
    dv = dinv_ref[...]
    y = dv * (a0_ref[...] + a1_ref[...] + hp_ref[...]) + b_ref[...]
    mean = jnp.sum(y, axis=0, keepdims=True) * (1.0 / N)
    var = jnp.sum(y * y, axis=0, keepdims=True) * (1.0 / N) - mean * mean
    z = g_ref[...] * (y - mean) * lax.rsqrt(var + EPS) + bt_ref[...]
    z = jnp.maximum(z, 0.0)
    h2 = lax.dot_general(z, w_ref[...], (((1,), (1,)), ((), ())),
                         preferred_element_type=jnp.float32)
    out_ref[...] = h2 * dv


def _final_body(a0_ref, a1_ref, hp_ref, dinv_ref, b_ref, g_ref, bt_ref, x_ref,
                out_ref):
    dv = dinv_ref[...]
    y = dv * (a0_ref[...] + a1_ref[...] + hp_ref[...]) + b_ref[...]
    mean = jnp.sum(y, axis=0, keepdims=True) * (1.0 / N)
    var = jnp.sum(y * y, axis=0, keepdims=True) * (1.0 / N) - mean * mean
    z = g_ref[...] * (y - mean) * lax.rsqrt(var + EPS) + bt_ref[...]
    out_ref[...] = jnp.maximum(z + x_ref[...], 0.0)


def _f32(*shapes):
    return [jax.ShapeDtypeStruct(s, jnp.float32) for s in shapes]


def kernel(x, edge_index, W1, b1, g1, bt1, W2, b2, g2, bt2):
    src = edge_index[0].astype(jnp.int32)
    dst = edge_index[1].astype(jnp.int32)
    npad = E_PAD - E
    pad_i = jnp.arange(npad, dtype=jnp.int32)
    src_p = jnp.concatenate([src, pad_i % N]).reshape(NW, NCHUNK, CH)
    dst_p = jnp.concatenate([dst, N + pad_i % (ACC_ROWS - N)]
                            ).reshape(NW, NCHUNK, CH)

    degp = _deg_kernel(dst_p)
    degf = degp.reshape(NC, DEG_ROWS * 128)[:, :N]
    d0 = degf[0][:, None]
    d1 = degf[1][:, None]

    h1p, dinv = pl.pallas_call(
        _mm_scale_body,
        out_shape=_f32((N, D), (N, 1)),
    )(x, W1, d0, d1)

    acc1 = _edge_scatter_kernel(h1p, src_p, dst_p)

    h2p = pl.pallas_call(
        _mid_body,
        out_shape=_f32((N, D))[0],
    )(acc1[0, :N, :], acc1[1, :N, :], h1p, dinv,
      b1[None, :], g1[None, :], bt1[None, :], W2)

    acc2 = _edge_scatter_kernel(h2p, src_p, dst_p)

    out = pl.pallas_call(
        _final_body,
        out_shape=_f32((N, D))[0],
    )(acc2[0, :N, :], acc2[1, :N, :], h2p, dinv,
      b2[None, :], g2[None, :], bt2[None, :], x)
    return out


# speedup vs baseline: 33.0021x; 1.1004x over previous
import dataclasses
import functools

import jax
import jax.numpy as jnp
from jax import lax
from jax.experimental import pallas as pl
from jax.experimental.pallas import tpu as pltpu
from jax.experimental.pallas import tpu_sc as plsc

N = 10000
D = 128
E = 320000
EPS = 1e-5

NC = 2
NS = 16
NW = NC * NS

CH = 120
NCHUNK = 84
E_PW = NCHUNK * CH
E_PAD = E_PW * NW
ACC_ROWS = 10112
RPT = ACC_ROWS // NS
DEG_ROWS = 128

_mesh = plsc.VectorSubcoreMesh(core_axis_name="c", subcore_axis_name="s")

_sc_params = pltpu.CompilerParams()
if "needs_layout_passes" in pltpu.CompilerParams.__dataclass_fields__:
    _sc_params = dataclasses.replace(_sc_params, needs_layout_passes=False)


@functools.partial(
    pl.kernel,
    out_type=jax.ShapeDtypeStruct((NC, DEG_ROWS, 128), jnp.float32),
    mesh=_mesh,
    compiler_params=_sc_params,
    scratch_types=[
        pltpu.VMEM((E_PW,), jnp.int32),
        pltpu.VMEM((DEG_ROWS, 128), jnp.float32),
        pltpu.VMEM((DEG_ROWS,), jnp.int32),
        pltpu.VMEM((DEG_ROWS // NS, 128), jnp.float32),
        pltpu.VMEM_SHARED((DEG_ROWS, 128), jnp.float32),
        pltpu.SemaphoreType.DMA,
    ],
)
def _deg_kernel(dst_hbm, out_hbm, dst_v, hist_v, rowidx_v, stage_v, acc_sh,
                sem):
    c = lax.axis_index("c")
    s = lax.axis_index("s")
    wid = c * NS + s
    zeros16 = jnp.zeros((16,), jnp.float32)
    ones16 = jnp.ones((16,), jnp.float32)

    cp = pltpu.async_copy(dst_hbm.at[wid], dst_v, sem)

    @pl.loop(0, DEG_ROWS)
    def _(r):
        @pl.loop(0, 8)
        def _(j):
            hist_v[r, pl.ds(j * 16, 16)] = zeros16

    @pl.loop(0, DEG_ROWS // 16)
    def _(j):
        rowidx_v[pl.ds(j * 16, 16)] = lax.iota(jnp.int32, 16) + j * 16

    cp.wait()

    @pl.loop(0, E_PW // 16)
    def _(i):
        idx = dst_v[pl.ds(i * 16, 16)]
        q = lax.shift_right_logical(idx, 7)
        r = lax.bitwise_and(idx, 127)
        plsc.addupdate_scatter(hist_v, [q, r], ones16)

    @pl.when(s == 0)
    def _():
        pltpu.sync_copy(hist_v, acc_sh)

    plsc.subcore_barrier()

    @pl.when(s != 0)
    def _():
        pltpu.sync_copy(hist_v, acc_sh.at[rowidx_v], add=True)

    plsc.subcore_barrier()
    rpt = DEG_ROWS // NS
    pltpu.sync_copy(acc_sh.at[pl.ds(s * rpt, rpt), :], stage_v)
    pltpu.sync_copy(stage_v, out_hbm.at[c, pl.ds(s * rpt, rpt), :])


@functools.partial(
    pl.kernel,
    out_type=jax.ShapeDtypeStruct((NC, ACC_ROWS, D), jnp.float32),
    mesh=_mesh,
    compiler_params=_sc_params,
    scratch_types=[
        pltpu.VMEM((4, CH), jnp.int32),
        pltpu.VMEM((4, CH), jnp.int32),
        pltpu.VMEM((CH, D), jnp.float32),
        pltpu.VMEM((CH, D), jnp.float32),
        pltpu.VMEM((CH, D), jnp.float32),
        pltpu.VMEM_SHARED((ACC_ROWS, D), jnp.float32),
        pltpu.SemaphoreType.DMA,
        pltpu.SemaphoreType.DMA,
        pltpu.SemaphoreType.DMA,
        pltpu.SemaphoreType.DMA,
        pltpu.SemaphoreType.DMA,
        pltpu.SemaphoreType.DMA,
        pltpu.SemaphoreType.DMA,
    ],
)
def _edge_scatter_kernel(table_hbm, src_hbm, dst_hbm, out_hbm,
                         sidx_v, didx_v, rows0_v, rows1_v, rows2_v, acc_sh,
                         semg0, semg1, semg2, semi0, semi1, semi2, semi3):
    c = lax.axis_index("c")
    s = lax.axis_index("s")
    wid = c * NS + s
    zeros16 = jnp.zeros((16,), jnp.float32)
    semg = (semg0, semg1, semg2)
    semi = (semi0, semi1, semi2, semi3)
    rows = (rows0_v, rows1_v, rows2_v)

    def idx_load(k, q):
        pltpu.async_copy(src_hbm.at[wid, k], sidx_v.at[q], semi[q])
        pltpu.async_copy(dst_hbm.at[wid, k], didx_v.at[q], semi[q])

    def idx_wait(k, q):
        pltpu.make_async_copy(src_hbm.at[wid, k], sidx_v.at[q],
                              semi[q]).wait()
        pltpu.make_async_copy(dst_hbm.at[wid, k], didx_v.at[q],
                              semi[q]).wait()

    def gather_start(q, p):
        pltpu.async_copy(table_hbm.at[sidx_v.at[q]], rows[p], semg[p])

    def gather_wait(q, p):
        pltpu.make_async_copy(table_hbm.at[sidx_v.at[q]], rows[p],
                              semg[p]).wait()

    for q in range(4):
        idx_load(q, q)

    @pl.loop(0, CH)
    def _(r):
        @pl.loop(0, D // 16)
        def _(j):
            rows0_v[r, pl.ds(j * 16, 16)] = zeros16

    for j in range(5):
        pltpu.sync_copy(rows0_v, acc_sh.at[pl.ds(s * RPT + j * CH, CH), :])
    pltpu.sync_copy(rows0_v.at[pl.ds(0, 32), :],
                    acc_sh.at[pl.ds(s * RPT + 5 * CH, 32), :])

    plsc.subcore_barrier()

    idx_wait(0, 0)
    gather_start(0, 0)
    idx_wait(1, 1)
    gather_start(1, 1)

    @pl.loop(0, NCHUNK // 12)
    def _(t):
        k0 = 12 * t
        for u in range(12):
            k = k0 + u
            q = u % 4
            p = u % 3

            @pl.when(k + 2 < NCHUNK)
            def _():
                idx_wait(k + 2, (q + 2) % 4)
                gather_start((q + 2) % 4, (p + 2) % 3)

            gather_wait(q, p)
            pltpu.sync_copy(rows[p], acc_sh.at[didx_v.at[q]], add=True)

            @pl.when(k + 4 < NCHUNK)
            def _():
                idx_load(k + 4, q)

    plsc.subcore_barrier()

    for j in range(5):
        pltpu.sync_copy(acc_sh.at[pl.ds(s * RPT + j * CH, CH), :], rows0_v)
        pltpu.sync_copy(rows0_v,
                        out_hbm.at[c, pl.ds(s * RPT + j * CH, CH), :])
    pltpu.sync_copy(acc_sh.at[pl.ds(s * RPT + 5 * CH, 32), :],
                    rows0_v.at[pl.ds(0, 32), :])
    pltpu.sync_copy(rows0_v.at[pl.ds(0, 32), :],
                    out_hbm.at[c, pl.ds(s * RPT + 5 * CH, 32), :])


def _mm_scale_body(x_ref, w_ref, d0_ref, d1_ref, hp_ref, dinv_ref):
    deg = d0_ref[...] + d1_ref[...] + 1.0
    dinv = lax.rsqrt(deg)
    dinv_ref[...] = dinv
    h = lax.dot_general(x_ref[...], w_ref[...], (((1,), (1,)), ((), ())),
                        preferred_element_type=jnp.float32)
    hp_ref[...] = h * dinv


def _mid_body(a0_ref, a1_ref, hp_ref, dinv_ref, b_ref, g_ref, bt_ref, w_ref,
              out_ref):
    dv = dinv_ref[...]
    y = dv * (a0_ref[...] + a1_ref[...] + hp_ref[...]) + b_ref[...]
    mean = jnp.sum(y, axis=0, keepdims=True) * (1.0 / N)
    var = jnp.sum(y * y, axis=0, keepdims=True) * (1.0 / N) - mean * mean
    z = g_ref[...] * (y - mean) * lax.rsqrt(var + EPS) + bt_ref[...]
    z = jnp.maximum(z, 0.0)
    h2 = lax.dot_general(z, w_ref[...], (((1,), (1,)), ((), ())),
                         preferred_element_type=jnp.float32)
    out_ref[...] = h2 * dv


def _final_body(a0_ref, a1_ref, hp_ref, dinv_ref, b_ref, g_ref, bt_ref, x_ref,
                out_ref):
    dv = dinv_ref[...]
    y = dv * (a0_ref[...] + a1_ref[...] + hp_ref[...]) + b_ref[...]
    mean = jnp.sum(y, axis=0, keepdims=True) * (1.0 / N)
    var = jnp.sum(y * y, axis=0, keepdims=True) * (1.0 / N) - mean * mean
    z = g_ref[...] * (y - mean) * lax.rsqrt(var + EPS) + bt_ref[...]
    out_ref[...] = jnp.maximum(z + x_ref[...], 0.0)


def _f32(*shapes):
    return [jax.ShapeDtypeStruct(s, jnp.float32) for s in shapes]


def kernel(x, edge_index, W1, b1, g1, bt1, W2, b2, g2, bt2):
    src = edge_index[0].astype(jnp.int32)
    dst = edge_index[1].astype(jnp.int32)
    npad = E_PAD - E
    pad_i = jnp.arange(npad, dtype=jnp.int32)
    src_p = jnp.concatenate([src, pad_i % N]).reshape(NW, NCHUNK, CH)
    dst_p = jnp.concatenate([dst, N + pad_i % (ACC_ROWS - N)]
                            ).reshape(NW, NCHUNK, CH)

    degp = _deg_kernel(dst_p.reshape(NW, E_PW))
    degf = degp.reshape(NC, DEG_ROWS * 128)[:, :N]
    d0 = degf[0][:, None]
    d1 = degf[1][:, None]

    h1p, dinv = pl.pallas_call(
        _mm_scale_body,
        out_shape=_f32((N, D), (N, 1)),
    )(x, W1, d0, d1)

    acc1 = _edge_scatter_kernel(h1p, src_p, dst_p)

    h2p = pl.pallas_call(
        _mid_body,
        out_shape=_f32((N, D))[0],
    )(acc1[0, :N, :], acc1[1, :N, :], h1p, dinv,
      b1[None, :], g1[None, :], bt1[None, :], W2)

    acc2 = _edge_scatter_kernel(h2p, src_p, dst_p)

    out = pl.pallas_call(
        _final_body,
        out_shape=_f32((N, D))[0],
    )(acc2[0, :N, :], acc2[1, :N, :], h2p, dinv,
      b2[None, :], g2[None, :], bt2[None, :], x)
    return out


# speedup vs baseline: 36.5599x; 1.1078x over previous
import dataclasses
import functools

import jax
import jax.numpy as jnp
from jax import lax
from jax.experimental import pallas as pl
from jax.experimental.pallas import tpu as pltpu
from jax.experimental.pallas import tpu_sc as plsc

N = 10000
D = 128
E = 320000
EPS = 1e-5

NC = 2
NS = 16
NW = NC * NS

CH = 120
NCHUNK = 84
E_PW = NCHUNK * CH
E_PAD = E_PW * NW
ACC_ROWS = 10112
RPT = ACC_ROWS // NS
DEG_ROWS = 128

_mesh = plsc.VectorSubcoreMesh(core_axis_name="c", subcore_axis_name="s")

_sc_params = pltpu.CompilerParams()
if "needs_layout_passes" in pltpu.CompilerParams.__dataclass_fields__:
    _sc_params = dataclasses.replace(_sc_params, needs_layout_passes=False)


@functools.partial(
    pl.kernel,
    out_type=jax.ShapeDtypeStruct((NC, DEG_ROWS, 128), jnp.float32),
    mesh=_mesh,
    compiler_params=_sc_params,
    scratch_types=[
        pltpu.VMEM((E_PW,), jnp.int32),
        pltpu.VMEM((DEG_ROWS, 128), jnp.float32),
        pltpu.VMEM((DEG_ROWS,), jnp.int32),
        pltpu.VMEM((DEG_ROWS // NS, 128), jnp.float32),
        pltpu.VMEM_SHARED((DEG_ROWS, 128), jnp.float32),
        pltpu.SemaphoreType.DMA,
    ],
)
def _deg_kernel(dst_hbm, out_hbm, dst_v, hist_v, rowidx_v, stage_v, acc_sh,
                sem):
    c = lax.axis_index("c")
    s = lax.axis_index("s")
    wid = c * NS + s
    zeros16 = jnp.zeros((16,), jnp.float32)
    ones16 = jnp.ones((16,), jnp.float32)

    cp = pltpu.async_copy(dst_hbm.at[wid], dst_v, sem)

    @pl.loop(0, DEG_ROWS)
    def _(r):
        @pl.loop(0, 8)
        def _(j):
            hist_v[r, pl.ds(j * 16, 16)] = zeros16

    @pl.loop(0, DEG_ROWS // 16)
    def _(j):
        rowidx_v[pl.ds(j * 16, 16)] = lax.iota(jnp.int32, 16) + j * 16

    cp.wait()

    @pl.loop(0, E_PW // 16)
    def _(i):
        idx = dst_v[pl.ds(i * 16, 16)]
        q = lax.shift_right_logical(idx, 7)
        r = lax.bitwise_and(idx, 127)
        plsc.addupdate_scatter(hist_v, [q, r], ones16)

    @pl.when(s == 0)
    def _():
        pltpu.sync_copy(hist_v, acc_sh)

    plsc.subcore_barrier()

    @pl.when(s != 0)
    def _():
        pltpu.sync_copy(hist_v, acc_sh.at[rowidx_v], add=True)

    plsc.subcore_barrier()
    rpt = DEG_ROWS // NS
    pltpu.sync_copy(acc_sh.at[pl.ds(s * rpt, rpt), :], stage_v)
    pltpu.sync_copy(stage_v, out_hbm.at[c, pl.ds(s * rpt, rpt), :])


@functools.partial(
    pl.kernel,
    out_type=jax.ShapeDtypeStruct((NC, ACC_ROWS, D), jnp.float32),
    mesh=_mesh,
    compiler_params=_sc_params,
    scratch_types=[
        pltpu.VMEM((4, CH), jnp.int32),
        pltpu.VMEM((4, CH), jnp.int32),
        pltpu.VMEM((CH, D), jnp.float32),
        pltpu.VMEM((CH, D), jnp.float32),
        pltpu.VMEM((CH, D), jnp.float32),
        pltpu.VMEM_SHARED((ACC_ROWS, D), jnp.float32),
        pltpu.SemaphoreType.DMA,
        pltpu.SemaphoreType.DMA,
        pltpu.SemaphoreType.DMA,
        pltpu.SemaphoreType.DMA,
        pltpu.SemaphoreType.DMA,
        pltpu.SemaphoreType.DMA,
        pltpu.SemaphoreType.DMA,
    ],
)
def _edge_scatter_kernel(table_hbm, src_hbm, dst_hbm, out_hbm,
                         sidx_v, didx_v, rows0_v, rows1_v, rows2_v, acc_sh,
                         semg0, semg1, semg2, semi0, semi1, semi2, semi3):
    c = lax.axis_index("c")
    s = lax.axis_index("s")
    wid = c * NS + s
    zeros16 = jnp.zeros((16,), jnp.float32)
    semg = (semg0, semg1, semg2)
    semi = (semi0, semi1, semi2, semi3)
    rows = (rows0_v, rows1_v, rows2_v)

    def idx_load(k, q):
        pltpu.async_copy(src_hbm.at[wid, k], sidx_v.at[q], semi[q])
        pltpu.async_copy(dst_hbm.at[wid, k], didx_v.at[q], semi[q])

    def idx_wait(k, q):
        pltpu.make_async_copy(src_hbm.at[wid, k], sidx_v.at[q],
                              semi[q]).wait()
        pltpu.make_async_copy(dst_hbm.at[wid, k], didx_v.at[q],
                              semi[q]).wait()

    def gather_start(q, p):
        pltpu.async_copy(table_hbm.at[sidx_v.at[q]], rows[p], semg[p])

    def gather_wait(q, p):
        pltpu.make_async_copy(table_hbm.at[sidx_v.at[q]], rows[p],
                              semg[p]).wait()

    for q in range(4):
        idx_load(q, q)

    @pl.loop(0, CH)
    def _(r):
        @pl.loop(0, D // 16)
        def _(j):
            rows0_v[r, pl.ds(j * 16, 16)] = zeros16

    for j in range(5):
        pltpu.sync_copy(rows0_v, acc_sh.at[pl.ds(s * RPT + j * CH, CH), :])
    pltpu.sync_copy(rows0_v.at[pl.ds(0, 32), :],
                    acc_sh.at[pl.ds(s * RPT + 5 * CH, 32), :])

    plsc.subcore_barrier()

    idx_wait(0, 0)
    gather_start(0, 0)
    idx_wait(1, 1)
    gather_start(1, 1)

    @pl.loop(0, NCHUNK // 12)
    def _(t):
        k0 = 12 * t
        for u in range(12):
            k = k0 + u
            q = u % 4
            p = u % 3

            @pl.when(k + 2 < NCHUNK)
            def _():
                idx_wait(k + 2, (q + 2) % 4)
                gather_start((q + 2) % 4, (p + 2) % 3)

            gather_wait(q, p)
            pltpu.sync_copy(rows[p], acc_sh.at[didx_v.at[q]], add=True)

            @pl.when(k + 4 < NCHUNK)
            def _():
                idx_load(k + 4, q)

    plsc.subcore_barrier()

    for j in range(5):
        pltpu.sync_copy(acc_sh.at[pl.ds(s * RPT + j * CH, CH), :], rows0_v)
        pltpu.sync_copy(rows0_v,
                        out_hbm.at[c, pl.ds(s * RPT + j * CH, CH), :])
    pltpu.sync_copy(acc_sh.at[pl.ds(s * RPT + 5 * CH, 32), :],
                    rows0_v.at[pl.ds(0, 32), :])
    pltpu.sync_copy(rows0_v.at[pl.ds(0, 32), :],
                    out_hbm.at[c, pl.ds(s * RPT + 5 * CH, 32), :])


_NBLK = N // 128
_NTAIL = N - _NBLK * 128


def _dinv_t(deg_ref):
    deg = deg_ref[0] + deg_ref[1] + 1.0
    return lax.transpose(lax.rsqrt(deg), (1, 0))


def _scale_rows(v, dvt):
    blocks = []
    for b in range(_NBLK):
        col = lax.slice(dvt, (0, b), (128, b + 1))
        blocks.append(v[b * 128:(b + 1) * 128, :] * col)
    col = lax.slice(dvt, (0, _NBLK), (_NTAIL, _NBLK + 1))
    blocks.append(v[_NBLK * 128:N, :] * col)
    return jnp.concatenate(blocks, axis=0)


def _mm_scale_body(x_ref, w_ref, deg_ref, hp_ref):
    dvt = _dinv_t(deg_ref)
    h = lax.dot_general(x_ref[...], w_ref[...], (((1,), (1,)), ((), ())),
                        preferred_element_type=jnp.float32)
    hp_ref[...] = _scale_rows(h, dvt)


def _mid_body(acc_ref, hp_ref, deg_ref, b_ref, g_ref, bt_ref, w_ref,
              out_ref):
    dvt = _dinv_t(deg_ref)
    acc = acc_ref[0, :N, :] + acc_ref[1, :N, :]
    y = _scale_rows(acc + hp_ref[...], dvt) + b_ref[...]
    mean = jnp.sum(y, axis=0, keepdims=True) * (1.0 / N)
    var = jnp.sum(y * y, axis=0, keepdims=True) * (1.0 / N) - mean * mean
    z = g_ref[...] * (y - mean) * lax.rsqrt(var + EPS) + bt_ref[...]
    z = jnp.maximum(z, 0.0)
    h2 = lax.dot_general(z, w_ref[...], (((1,), (1,)), ((), ())),
                         preferred_element_type=jnp.float32)
    out_ref[...] = _scale_rows(h2, dvt)


def _final_body(acc_ref, hp_ref, deg_ref, b_ref, g_ref, bt_ref, x_ref,
                out_ref):
    dvt = _dinv_t(deg_ref)
    acc = acc_ref[0, :N, :] + acc_ref[1, :N, :]
    y = _scale_rows(acc + hp_ref[...], dvt) + b_ref[...]
    mean = jnp.sum(y, axis=0, keepdims=True) * (1.0 / N)
    var = jnp.sum(y * y, axis=0, keepdims=True) * (1.0 / N) - mean * mean
    z = g_ref[...] * (y - mean) * lax.rsqrt(var + EPS) + bt_ref[...]
    out_ref[...] = jnp.maximum(z + x_ref[...], 0.0)


def _f32(*shapes):
    return [jax.ShapeDtypeStruct(s, jnp.float32) for s in shapes]


def kernel(x, edge_index, W1, b1, g1, bt1, W2, b2, g2, bt2):
    src = edge_index[0].astype(jnp.int32)
    dst = edge_index[1].astype(jnp.int32)
    npad = E_PAD - E
    pad_i = jnp.arange(npad, dtype=jnp.int32)
    src_p = jnp.concatenate([src, pad_i % N]).reshape(NW, NCHUNK, CH)
    dst_p = jnp.concatenate([dst, N + pad_i % (ACC_ROWS - N)]
                            ).reshape(NW, NCHUNK, CH)

    degp = _deg_kernel(dst_p.reshape(NW, E_PW))

    h1p = pl.pallas_call(
        _mm_scale_body,
        out_shape=_f32((N, D))[0],
    )(x, W1, degp)

    acc1 = _edge_scatter_kernel(h1p, src_p, dst_p)

    h2p = pl.pallas_call(
        _mid_body,
        out_shape=_f32((N, D))[0],
    )(acc1, h1p, degp, b1[None, :], g1[None, :], bt1[None, :], W2)

    acc2 = _edge_scatter_kernel(h2p, src_p, dst_p)

    out = pl.pallas_call(
        _final_body,
        out_shape=_f32((N, D))[0],
    )(acc2, h2p, degp, b2[None, :], g2[None, :], bt2[None, :], x)
    return out


# speedup vs baseline: 36.6062x; 1.0013x over previous
import dataclasses
import functools

import jax
import jax.numpy as jnp
from jax import lax
from jax.experimental import pallas as pl
from jax.experimental.pallas import tpu as pltpu
from jax.experimental.pallas import tpu_sc as plsc

N = 10000
D = 128
E = 320000
EPS = 1e-5

NC = 2
NS = 16
NW = NC * NS

CH = 120
NCHUNK = 84
E_PW = NCHUNK * CH
E_PAD = E_PW * NW
ACC_ROWS = 10112
RPT = ACC_ROWS // NS
DEG_ROWS = 128

_mesh = plsc.VectorSubcoreMesh(core_axis_name="c", subcore_axis_name="s")

_sc_params = pltpu.CompilerParams()
if "needs_layout_passes" in pltpu.CompilerParams.__dataclass_fields__:
    _sc_params = dataclasses.replace(_sc_params, needs_layout_passes=False)


@functools.partial(
    pl.kernel,
    out_type=jax.ShapeDtypeStruct((NC, DEG_ROWS, 128), jnp.float32),
    mesh=_mesh,
    compiler_params=_sc_params,
    scratch_types=[
        pltpu.VMEM((E_PW,), jnp.int32),
        pltpu.VMEM((DEG_ROWS, 128), jnp.float32),
        pltpu.VMEM((DEG_ROWS,), jnp.int32),
        pltpu.VMEM((DEG_ROWS // NS, 128), jnp.float32),
        pltpu.VMEM_SHARED((DEG_ROWS, 128), jnp.float32),
        pltpu.SemaphoreType.DMA,
    ],
)
def _deg_kernel(dst_hbm, out_hbm, dst_v, hist_v, rowidx_v, stage_v, acc_sh,
                sem):
    c = lax.axis_index("c")
    s = lax.axis_index("s")
    wid = c * NS + s
    zeros16 = jnp.zeros((16,), jnp.float32)
    ones16 = jnp.ones((16,), jnp.float32)

    cp = pltpu.async_copy(dst_hbm.at[wid], dst_v, sem)

    @pl.loop(0, DEG_ROWS)
    def _(r):
        @pl.loop(0, 8)
        def _(j):
            hist_v[r, pl.ds(j * 16, 16)] = zeros16

    @pl.loop(0, DEG_ROWS // 16)
    def _(j):
        rowidx_v[pl.ds(j * 16, 16)] = lax.iota(jnp.int32, 16) + j * 16

    cp.wait()

    @pl.loop(0, E_PW // 16)
    def _(i):
        idx = dst_v[pl.ds(i * 16, 16)]
        q = lax.shift_right_logical(idx, 7)
        r = lax.bitwise_and(idx, 127)
        plsc.addupdate_scatter(hist_v, [q, r], ones16)

    @pl.when(s == 0)
    def _():
        pltpu.sync_copy(hist_v, acc_sh)

    plsc.subcore_barrier()

    @pl.when(s != 0)
    def _():
        pltpu.sync_copy(hist_v, acc_sh.at[rowidx_v], add=True)

    plsc.subcore_barrier()
    rpt = DEG_ROWS // NS
    pltpu.sync_copy(acc_sh.at[pl.ds(s * rpt, rpt), :], stage_v)
    pltpu.sync_copy(stage_v, out_hbm.at[c, pl.ds(s * rpt, rpt), :])


@functools.partial(
    pl.kernel,
    out_type=jax.ShapeDtypeStruct((NC, ACC_ROWS, D), jnp.float32),
    mesh=_mesh,
    compiler_params=_sc_params,
    scratch_types=[
        pltpu.VMEM((4, CH), jnp.int32),
        pltpu.VMEM((4, CH), jnp.int32),
        pltpu.VMEM((CH, D), jnp.float32),
        pltpu.VMEM((CH, D), jnp.float32),
        pltpu.VMEM((CH, D), jnp.float32),
        pltpu.VMEM_SHARED((ACC_ROWS, D), jnp.float32),
        pltpu.SemaphoreType.DMA,
        pltpu.SemaphoreType.DMA,
        pltpu.SemaphoreType.DMA,
        pltpu.SemaphoreType.DMA,
        pltpu.SemaphoreType.DMA,
        pltpu.SemaphoreType.DMA,
        pltpu.SemaphoreType.DMA,
    ],
)
def _edge_scatter_kernel(table_hbm, src_hbm, dst_hbm, out_hbm,
                         sidx_v, didx_v, rows0_v, rows1_v, rows2_v, acc_sh,
                         semg0, semg1, semg2, semi0, semi1, semi2, semi3):
    c = lax.axis_index("c")
    s = lax.axis_index("s")
    wid = c * NS + s
    zeros16 = jnp.zeros((16,), jnp.float32)
    semg = (semg0, semg1, semg2)
    semi = (semi0, semi1, semi2, semi3)
    rows = (rows0_v, rows1_v, rows2_v)

    def idx_load(k, q):
        pltpu.async_copy(src_hbm.at[wid, k], sidx_v.at[q], semi[q])
        pltpu.async_copy(dst_hbm.at[wid, k], didx_v.at[q], semi[q])

    def idx_wait(k, q):
        pltpu.make_async_copy(src_hbm.at[wid, k], sidx_v.at[q],
                              semi[q]).wait()
        pltpu.make_async_copy(dst_hbm.at[wid, k], didx_v.at[q],
                              semi[q]).wait()

    def gather_start(q, p):
        pltpu.async_copy(table_hbm.at[sidx_v.at[q]], rows[p], semg[p])

    def gather_wait(q, p):
        pltpu.make_async_copy(table_hbm.at[sidx_v.at[q]], rows[p],
                              semg[p]).wait()

    for q in range(4):
        idx_load(q, q)

    @pl.loop(0, CH)
    def _(r):
        @pl.loop(0, D // 16)
        def _(j):
            rows0_v[r, pl.ds(j * 16, 16)] = zeros16

    for j in range(5):
        pltpu.sync_copy(rows0_v, acc_sh.at[pl.ds(s * RPT + j * CH, CH), :])
    pltpu.sync_copy(rows0_v.at[pl.ds(0, 32), :],
                    acc_sh.at[pl.ds(s * RPT + 5 * CH, 32), :])

    plsc.subcore_barrier()

    idx_wait(0, 0)
    gather_start(0, 0)
    idx_wait(1, 1)
    gather_start(1, 1)

    @pl.loop(0, NCHUNK // 12)
    def _(t):
        k0 = 12 * t
        for u in range(12):
            k = k0 + u
            q = u % 4
            p = u % 3

            @pl.when(k + 2 < NCHUNK)
            def _():
                idx_wait(k + 2, (q + 2) % 4)
                gather_start((q + 2) % 4, (p + 2) % 3)

            gather_wait(q, p)
            pltpu.sync_copy(rows[p], acc_sh.at[didx_v.at[q]], add=True)

            @pl.when(k + 4 < NCHUNK)
            def _():
                idx_load(k + 4, q)

    plsc.subcore_barrier()

    for j in range(5):
        pltpu.sync_copy(acc_sh.at[pl.ds(s * RPT + j * CH, CH), :], rows0_v)
        pltpu.sync_copy(rows0_v,
                        out_hbm.at[c, pl.ds(s * RPT + j * CH, CH), :])
    pltpu.sync_copy(acc_sh.at[pl.ds(s * RPT + 5 * CH, 32), :],
                    rows0_v.at[pl.ds(0, 32), :])
    pltpu.sync_copy(rows0_v.at[pl.ds(0, 32), :],
                    out_hbm.at[c, pl.ds(s * RPT + 5 * CH, 32), :])


_NBLK = N // 128
_NTAIL = N - _NBLK * 128


def _dinv_t(deg_ref):
    deg = deg_ref[:DEG_ROWS, :] + deg_ref[DEG_ROWS:, :] + 1.0
    return lax.transpose(lax.rsqrt(deg), (1, 0))


def _scale_rows(v, dvt):
    blocks = []
    for b in range(_NBLK):
        col = lax.slice(dvt, (0, b), (128, b + 1))
        blocks.append(v[b * 128:(b + 1) * 128, :] * col)
    col = lax.slice(dvt, (0, _NBLK), (_NTAIL, _NBLK + 1))
    blocks.append(v[_NBLK * 128:N, :] * col)
    return jnp.concatenate(blocks, axis=0)


def _mm_scale_body(x_ref, w_ref, deg_ref, hp_ref):
    dvt = _dinv_t(deg_ref)
    h = lax.dot_general(x_ref[...], w_ref[...], (((1,), (1,)), ((), ())),
                        preferred_element_type=jnp.float32)
    hp_ref[...] = _scale_rows(h, dvt)


def _mid_body(acc_ref, hp_ref, deg_ref, b_ref, g_ref, bt_ref, w_ref,
              out_ref):
    dvt = _dinv_t(deg_ref)
    acc = acc_ref[:N, :] + acc_ref[ACC_ROWS:ACC_ROWS + N, :]
    y = _scale_rows(acc + hp_ref[...], dvt) + b_ref[...]
    mean = jnp.sum(y, axis=0, keepdims=True) * (1.0 / N)
    var = jnp.sum(y * y, axis=0, keepdims=True) * (1.0 / N) - mean * mean
    z = g_ref[...] * (y - mean) * lax.rsqrt(var + EPS) + bt_ref[...]
    z = jnp.maximum(z, 0.0)
    h2 = lax.dot_general(z, w_ref[...], (((1,), (1,)), ((), ())),
                         preferred_element_type=jnp.float32)
    out_ref[...] = _scale_rows(h2, dvt)


def _final_body(acc_ref, hp_ref, deg_ref, b_ref, g_ref, bt_ref, x_ref,
                out_ref):
    dvt = _dinv_t(deg_ref)
    acc = acc_ref[:N, :] + acc_ref[ACC_ROWS:ACC_ROWS + N, :]
    y = _scale_rows(acc + hp_ref[...], dvt) + b_ref[...]
    mean = jnp.sum(y, axis=0, keepdims=True) * (1.0 / N)
    var = jnp.sum(y * y, axis=0, keepdims=True) * (1.0 / N) - mean * mean
    z = g_ref[...] * (y - mean) * lax.rsqrt(var + EPS) + bt_ref[...]
    out_ref[...] = jnp.maximum(z + x_ref[...], 0.0)


def _f32(*shapes):
    return [jax.ShapeDtypeStruct(s, jnp.float32) for s in shapes]


def kernel(x, edge_index, W1, b1, g1, bt1, W2, b2, g2, bt2):
    src = edge_index[0].astype(jnp.int32)
    dst = edge_index[1].astype(jnp.int32)
    npad = E_PAD - E
    pad_i = jnp.arange(npad, dtype=jnp.int32)
    src_p = jnp.concatenate([src, pad_i % N]).reshape(NW, NCHUNK, CH)
    dst_p = jnp.concatenate([dst, N + pad_i % (ACC_ROWS - N)]
                            ).reshape(NW, NCHUNK, CH)

    degp = _deg_kernel(dst_p.reshape(NW, E_PW))
    deg2 = degp.reshape(NC * DEG_ROWS, 128)

    h1p = pl.pallas_call(
        _mm_scale_body,
        out_shape=_f32((N, D))[0],
    )(x, W1, deg2)

    acc1 = _edge_scatter_kernel(h1p, src_p, dst_p)

    h2p = pl.pallas_call(
        _mid_body,
        out_shape=_f32((N, D))[0],
    )(acc1.reshape(NC * ACC_ROWS, D), h1p, deg2,
      b1[None, :], g1[None, :], bt1[None, :], W2)

    acc2 = _edge_scatter_kernel(h2p, src_p, dst_p)

    out = pl.pallas_call(
        _final_body,
        out_shape=_f32((N, D))[0],
    )(acc2.reshape(NC * ACC_ROWS, D), h2p, deg2,
      b2[None, :], g2[None, :], bt2[None, :], x)
    return out


# speedup vs baseline: 37.0719x; 1.0127x over previous
import dataclasses
import functools

import jax
import jax.numpy as jnp
from jax import lax
from jax.experimental import pallas as pl
from jax.experimental.pallas import tpu as pltpu
from jax.experimental.pallas import tpu_sc as plsc

N = 10000
D = 128
E = 320000
EPS = 1e-5

NC = 2
NS = 16
NW = NC * NS

CH = 80
NCHUNK = 125
E_PW = NCHUNK * CH
ACC_ROWS = 10112
RPT = ACC_ROWS // NS
DEG_ROWS = 128

_mesh = plsc.VectorSubcoreMesh(core_axis_name="c", subcore_axis_name="s")

_sc_params = pltpu.CompilerParams()
if "needs_layout_passes" in pltpu.CompilerParams.__dataclass_fields__:
    _sc_params = dataclasses.replace(_sc_params, needs_layout_passes=False)


@functools.partial(
    pl.kernel,
    out_type=jax.ShapeDtypeStruct((NC, DEG_ROWS, 128), jnp.float32),
    mesh=_mesh,
    compiler_params=_sc_params,
    scratch_types=[
        pltpu.VMEM((E_PW,), jnp.int32),
        pltpu.VMEM((DEG_ROWS, 128), jnp.float32),
        pltpu.VMEM((DEG_ROWS,), jnp.int32),
        pltpu.VMEM((DEG_ROWS // NS, 128), jnp.float32),
        pltpu.VMEM_SHARED((DEG_ROWS, 128), jnp.float32),
        pltpu.SemaphoreType.DMA,
    ],
)
def _deg_kernel(dst_hbm, out_hbm, dst_v, hist_v, rowidx_v, stage_v, acc_sh,
                sem):
    c = lax.axis_index("c")
    s = lax.axis_index("s")
    wid = c * NS + s
    zeros16 = jnp.zeros((16,), jnp.float32)
    ones16 = jnp.ones((16,), jnp.float32)

    cp = pltpu.async_copy(dst_hbm.at[pl.ds(wid * E_PW, E_PW)], dst_v, sem)

    @pl.loop(0, DEG_ROWS)
    def _(r):
        @pl.loop(0, 8)
        def _(j):
            hist_v[r, pl.ds(j * 16, 16)] = zeros16

    @pl.loop(0, DEG_ROWS // 16)
    def _(j):
        rowidx_v[pl.ds(j * 16, 16)] = lax.iota(jnp.int32, 16) + j * 16

    cp.wait()

    @pl.loop(0, E_PW // 16)
    def _(i):
        idx = dst_v[pl.ds(i * 16, 16)]
        q = lax.shift_right_logical(idx, 7)
        r = lax.bitwise_and(idx, 127)
        plsc.addupdate_scatter(hist_v, [q, r], ones16)

    @pl.when(s == 0)
    def _():
        pltpu.sync_copy(hist_v, acc_sh)

    plsc.subcore_barrier()

    @pl.when(s != 0)
    def _():
        pltpu.sync_copy(hist_v, acc_sh.at[rowidx_v], add=True)

    plsc.subcore_barrier()
    rpt = DEG_ROWS // NS
    pltpu.sync_copy(acc_sh.at[pl.ds(s * rpt, rpt), :], stage_v)
    pltpu.sync_copy(stage_v, out_hbm.at[c, pl.ds(s * rpt, rpt), :])


@functools.partial(
    pl.kernel,
    out_type=jax.ShapeDtypeStruct((NC, ACC_ROWS, D), jnp.float32),
    mesh=_mesh,
    compiler_params=_sc_params,
    scratch_types=[
        pltpu.VMEM((8, CH), jnp.int32),
        pltpu.VMEM((8, CH), jnp.int32),
        pltpu.VMEM((CH, D), jnp.float32),
        pltpu.VMEM((CH, D), jnp.float32),
        pltpu.VMEM((CH, D), jnp.float32),
        pltpu.VMEM((CH, D), jnp.float32),
        pltpu.VMEM_SHARED((ACC_ROWS, D), jnp.float32),
    ] + [pltpu.SemaphoreType.DMA] * 12,
)
def _edge_scatter_kernel(table_hbm, src_hbm, dst_hbm, out_hbm,
                         sidx_v, didx_v, rows0_v, rows1_v, rows2_v, rows3_v,
                         acc_sh, *sems):
    c = lax.axis_index("c")
    s = lax.axis_index("s")
    wid = c * NS + s
    zeros16 = jnp.zeros((16,), jnp.float32)
    rows = (rows0_v, rows1_v, rows2_v, rows3_v)
    semg = sems[:4]
    semi = sems[4:]

    base = wid * E_PW

    def idx_load(k, q):
        pltpu.async_copy(src_hbm.at[pl.ds(base + k * CH, CH)], sidx_v.at[q],
                         semi[q])
        pltpu.async_copy(dst_hbm.at[pl.ds(base + k * CH, CH)], didx_v.at[q],
                         semi[q])

    def idx_wait(k, q):
        pltpu.make_async_copy(src_hbm.at[pl.ds(base + k * CH, CH)],
                              sidx_v.at[q], semi[q]).wait()
        pltpu.make_async_copy(dst_hbm.at[pl.ds(base + k * CH, CH)],
                              didx_v.at[q], semi[q]).wait()

    def gather_start(q, p):
        pltpu.async_copy(table_hbm.at[sidx_v.at[q]], rows[p], semg[p])

    def gather_wait(q, p):
        pltpu.make_async_copy(table_hbm.at[sidx_v.at[q]], rows[p],
                              semg[p]).wait()

    for q in range(8):
        idx_load(q, q)

    @pl.loop(0, CH)
    def _(r):
        @pl.loop(0, D // 16)
        def _(j):
            rows0_v[r, pl.ds(j * 16, 16)] = zeros16

    for j in range(7):
        pltpu.sync_copy(rows0_v, acc_sh.at[pl.ds(s * RPT + j * CH, CH), :])
    pltpu.sync_copy(rows0_v.at[pl.ds(0, RPT - 7 * CH), :],
                    acc_sh.at[pl.ds(s * RPT + 7 * CH, RPT - 7 * CH), :])

    plsc.subcore_barrier()

    for k in range(3):
        idx_wait(k, k)
        gather_start(k, k)

    def chunk_body(k, q, p, guard):
        def issue_next_gather():
            idx_wait(k + 3, (q + 3) % 8)
            gather_start((q + 3) % 8, (p + 3) % 4)

        def refill_idx():
            idx_load(k + 8, q)

        if guard:
            pl.when(k + 3 < NCHUNK)(issue_next_gather)
        gather_wait(q, p)
        pltpu.sync_copy(rows[p], acc_sh.at[didx_v.at[q]], add=True)
        if guard:
            pl.when(k + 8 < NCHUNK)(refill_idx)

    @pl.loop(0, (NCHUNK - 5) // 8)
    def _(t):
        k0 = 8 * t
        for u in range(8):
            chunk_body(k0 + u, u, u % 4, True)

    for k in range(NCHUNK - 5, NCHUNK):
        q = k % 8
        p = k % 4
        if k + 3 < NCHUNK:
            idx_wait(k + 3, (q + 3) % 8)
            gather_start((q + 3) % 8, (p + 3) % 4)
        gather_wait(q, p)
        pltpu.sync_copy(rows[p], acc_sh.at[didx_v.at[q]], add=True)

    plsc.subcore_barrier()

    for j in range(7):
        pltpu.sync_copy(acc_sh.at[pl.ds(s * RPT + j * CH, CH), :], rows0_v)
        pltpu.sync_copy(rows0_v,
                        out_hbm.at[c, pl.ds(s * RPT + j * CH, CH), :])
    tail = RPT - 7 * CH
    pltpu.sync_copy(acc_sh.at[pl.ds(s * RPT + 7 * CH, tail), :],
                    rows0_v.at[pl.ds(0, tail), :])
    pltpu.sync_copy(rows0_v.at[pl.ds(0, tail), :],
                    out_hbm.at[c, pl.ds(s * RPT + 7 * CH, tail), :])


_NBLK = N // 128
_NTAIL = N - _NBLK * 128


def _dinv_t(deg_ref):
    deg = deg_ref[:DEG_ROWS, :] + deg_ref[DEG_ROWS:, :] + 1.0
    return lax.transpose(lax.rsqrt(deg), (1, 0))


def _scale_rows(v, dvt):
    blocks = []
    for b in range(_NBLK):
        col = lax.slice(dvt, (0, b), (128, b + 1))
        blocks.append(v[b * 128:(b + 1) * 128, :] * col)
    col = lax.slice(dvt, (0, _NBLK), (_NTAIL, _NBLK + 1))
    blocks.append(v[_NBLK * 128:N, :] * col)
    return jnp.concatenate(blocks, axis=0)


def _mm_scale_body(x_ref, w_ref, deg_ref, hp_ref):
    dvt = _dinv_t(deg_ref)
    h = lax.dot_general(x_ref[...], w_ref[...], (((1,), (1,)), ((), ())),
                        preferred_element_type=jnp.float32)
    hp_ref[...] = _scale_rows(h, dvt)


def _mid_body(acc_ref, hp_ref, deg_ref, b_ref, g_ref, bt_ref, w_ref,
              out_ref):
    dvt = _dinv_t(deg_ref)
    acc = acc_ref[:N, :] + acc_ref[ACC_ROWS:ACC_ROWS + N, :]
    y = _scale_rows(acc + hp_ref[...], dvt) + b_ref[...]
    mean = jnp.sum(y, axis=0, keepdims=True) * (1.0 / N)
    var = jnp.sum(y * y, axis=0, keepdims=True) * (1.0 / N) - mean * mean
    z = g_ref[...] * (y - mean) * lax.rsqrt(var + EPS) + bt_ref[...]
    z = jnp.maximum(z, 0.0)
    h2 = lax.dot_general(z, w_ref[...], (((1,), (1,)), ((), ())),
                         preferred_element_type=jnp.float32)
    out_ref[...] = _scale_rows(h2, dvt)


def _final_body(acc_ref, hp_ref, deg_ref, b_ref, g_ref, bt_ref, x_ref,
                out_ref):
    dvt = _dinv_t(deg_ref)
    acc = acc_ref[:N, :] + acc_ref[ACC_ROWS:ACC_ROWS + N, :]
    y = _scale_rows(acc + hp_ref[...], dvt) + b_ref[...]
    mean = jnp.sum(y, axis=0, keepdims=True) * (1.0 / N)
    var = jnp.sum(y * y, axis=0, keepdims=True) * (1.0 / N) - mean * mean
    z = g_ref[...] * (y - mean) * lax.rsqrt(var + EPS) + bt_ref[...]
    out_ref[...] = jnp.maximum(z + x_ref[...], 0.0)


def _f32(*shapes):
    return [jax.ShapeDtypeStruct(s, jnp.float32) for s in shapes]


def kernel(x, edge_index, W1, b1, g1, bt1, W2, b2, g2, bt2):
    src_p = edge_index[0].astype(jnp.int32)
    dst_p = edge_index[1].astype(jnp.int32)

    degp = _deg_kernel(dst_p)
    deg2 = degp.reshape(NC * DEG_ROWS, 128)

    h1p = pl.pallas_call(
        _mm_scale_body,
        out_shape=_f32((N, D))[0],
    )(x, W1, deg2)

    acc1 = _edge_scatter_kernel(h1p, src_p, dst_p)

    h2p = pl.pallas_call(
        _mid_body,
        out_shape=_f32((N, D))[0],
    )(acc1.reshape(NC * ACC_ROWS, D), h1p, deg2,
      b1[None, :], g1[None, :], bt1[None, :], W2)

    acc2 = _edge_scatter_kernel(h2p, src_p, dst_p)

    out = pl.pallas_call(
        _final_body,
        out_shape=_f32((N, D))[0],
    )(acc2.reshape(NC * ACC_ROWS, D), h2p, deg2,
      b2[None, :], g2[None, :], bt2[None, :], x)
    return out


# speedup vs baseline: 37.0853x; 1.0004x over previous
import dataclasses
import functools

import jax
import jax.numpy as jnp
from jax import lax
from jax.experimental import pallas as pl
from jax.experimental.pallas import tpu as pltpu
from jax.experimental.pallas import tpu_sc as plsc

N = 10000
D = 128
E = 320000
EPS = 1e-5

NC = 2
NS = 16
NW = NC * NS

CH = 80
NCHUNK = 125
E_PW = NCHUNK * CH
ACC_ROWS = 10112
RPT = ACC_ROWS // NS
DEG_ROWS = 128

_mesh = plsc.VectorSubcoreMesh(core_axis_name="c", subcore_axis_name="s")

_sc_params = pltpu.CompilerParams()
if "needs_layout_passes" in pltpu.CompilerParams.__dataclass_fields__:
    _sc_params = dataclasses.replace(_sc_params, needs_layout_passes=False)


@functools.partial(
    pl.kernel,
    out_type=jax.ShapeDtypeStruct((NC, DEG_ROWS, 128), jnp.float32),
    mesh=_mesh,
    compiler_params=_sc_params,
    scratch_types=[
        pltpu.VMEM((E_PW,), jnp.int32),
        pltpu.VMEM((DEG_ROWS, 128), jnp.float32),
        pltpu.VMEM((DEG_ROWS,), jnp.int32),
        pltpu.VMEM((DEG_ROWS // NS, 128), jnp.float32),
        pltpu.VMEM_SHARED((DEG_ROWS, 128), jnp.float32),
        pltpu.SemaphoreType.DMA,
    ],
)
def _deg_kernel(dst_hbm, out_hbm, dst_v, hist_v, rowidx_v, stage_v, acc_sh,
                sem):
    c = lax.axis_index("c")
    s = lax.axis_index("s")
    wid = c * NS + s
    zeros16 = jnp.zeros((16,), jnp.float32)
    ones16 = jnp.ones((16,), jnp.float32)

    cp = pltpu.async_copy(dst_hbm.at[pl.ds(wid * E_PW, E_PW)], dst_v, sem)

    @pl.loop(0, DEG_ROWS)
    def _(r):
        @pl.loop(0, 8)
        def _(j):
            hist_v[r, pl.ds(j * 16, 16)] = zeros16

    @pl.loop(0, DEG_ROWS // 16)
    def _(j):
        rowidx_v[pl.ds(j * 16, 16)] = lax.iota(jnp.int32, 16) + j * 16

    cp.wait()

    @pl.loop(0, E_PW // 16)
    def _(i):
        idx = dst_v[pl.ds(i * 16, 16)]
        q = lax.shift_right_logical(idx, 7)
        r = lax.bitwise_and(idx, 127)
        plsc.addupdate_scatter(hist_v, [q, r], ones16)

    @pl.when(s == 0)
    def _():
        pltpu.sync_copy(hist_v, acc_sh)

    plsc.subcore_barrier()

    @pl.when(s != 0)
    def _():
        pltpu.sync_copy(hist_v, acc_sh.at[rowidx_v], add=True)

    plsc.subcore_barrier()
    rpt = DEG_ROWS // NS
    pltpu.sync_copy(acc_sh.at[pl.ds(s * rpt, rpt), :], stage_v)
    pltpu.sync_copy(stage_v, out_hbm.at[c, pl.ds(s * rpt, rpt), :])


@functools.partial(
    pl.kernel,
    out_type=jax.ShapeDtypeStruct((NC, ACC_ROWS, D), jnp.float32),
    mesh=_mesh,
    compiler_params=_sc_params,
    scratch_types=[
        pltpu.VMEM((8, CH), jnp.int32),
        pltpu.VMEM((8, CH), jnp.int32),
        pltpu.VMEM((CH, D), jnp.float32),
        pltpu.VMEM((CH, D), jnp.float32),
        pltpu.VMEM((CH, D), jnp.float32),
        pltpu.VMEM((CH, D), jnp.float32),
        pltpu.VMEM_SHARED((ACC_ROWS, D), jnp.float32),
    ] + [pltpu.SemaphoreType.DMA] * 12,
)
def _edge_scatter_kernel(table_hbm, src_hbm, dst_hbm, out_hbm,
                         sidx_v, didx_v, rows0_v, rows1_v, rows2_v, rows3_v,
                         acc_sh, *sems):
    c = lax.axis_index("c")
    s = lax.axis_index("s")
    wid = c * NS + s
    zeros16 = jnp.zeros((16,), jnp.float32)
    rows = (rows0_v, rows1_v, rows2_v, rows3_v)
    semg = sems[:4]
    semi = sems[4:]

    base = wid * E_PW

    def idx_load(k, q):
        pltpu.async_copy(src_hbm.at[pl.ds(base + k * CH, CH)], sidx_v.at[q],
                         semi[q])
        pltpu.async_copy(dst_hbm.at[pl.ds(base + k * CH, CH)], didx_v.at[q],
                         semi[q])

    def idx_wait(k, q):
        pltpu.make_async_copy(src_hbm.at[pl.ds(base + k * CH, CH)],
                              sidx_v.at[q], semi[q]).wait()
        pltpu.make_async_copy(dst_hbm.at[pl.ds(base + k * CH, CH)],
                              didx_v.at[q], semi[q]).wait()

    def gather_start(q, p):
        pltpu.async_copy(table_hbm.at[sidx_v.at[q]], rows[p], semg[p])

    def gather_wait(q, p):
        pltpu.make_async_copy(table_hbm.at[sidx_v.at[q]], rows[p],
                              semg[p]).wait()

    for q in range(8):
        idx_load(q, q)

    @pl.loop(0, CH)
    def _(r):
        @pl.loop(0, D // 16)
        def _(j):
            rows0_v[r, pl.ds(j * 16, 16)] = zeros16

    for j in range(7):
        pltpu.sync_copy(rows0_v, acc_sh.at[pl.ds(s * RPT + j * CH, CH), :])
    pltpu.sync_copy(rows0_v.at[pl.ds(0, RPT - 7 * CH), :],
                    acc_sh.at[pl.ds(s * RPT + 7 * CH, RPT - 7 * CH), :])

    plsc.subcore_barrier()

    for k in range(3):
        idx_wait(k, k)
        gather_start(k, k)

    def chunk_body(k, q, p, guard):
        def issue_next_gather():
            idx_wait(k + 3, (q + 3) % 8)
            gather_start((q + 3) % 8, (p + 3) % 4)

        def refill_idx():
            idx_load(k + 8, q)

        if guard:
            pl.when(k + 3 < NCHUNK)(issue_next_gather)
        gather_wait(q, p)
        pltpu.sync_copy(rows[p], acc_sh.at[didx_v.at[q]], add=True)
        if guard:
            pl.when(k + 8 < NCHUNK)(refill_idx)

    @pl.loop(0, (NCHUNK - 5) // 8)
    def _(t):
        k0 = 8 * t
        for u in range(8):
            chunk_body(k0 + u, u, u % 4, True)

    for k in range(NCHUNK - 5, NCHUNK):
        q = k % 8
        p = k % 4
        if k + 3 < NCHUNK:
            idx_wait(k + 3, (q + 3) % 8)
            gather_start((q + 3) % 8, (p + 3) % 4)
        gather_wait(q, p)
        pltpu.sync_copy(rows[p], acc_sh.at[didx_v.at[q]], add=True)

    plsc.subcore_barrier()

    for j in range(7):
        pltpu.sync_copy(acc_sh.at[pl.ds(s * RPT + j * CH, CH), :], rows0_v)
        pltpu.sync_copy(rows0_v,
                        out_hbm.at[c, pl.ds(s * RPT + j * CH, CH), :])
    tail = RPT - 7 * CH
    pltpu.sync_copy(acc_sh.at[pl.ds(s * RPT + 7 * CH, tail), :],
                    rows0_v.at[pl.ds(0, tail), :])
    pltpu.sync_copy(rows0_v.at[pl.ds(0, tail), :],
                    out_hbm.at[c, pl.ds(s * RPT + 7 * CH, tail), :])


_NBLK = N // 128
_NTAIL = N - _NBLK * 128


def _dinv_t(deg_ref):
    deg = deg_ref[:DEG_ROWS, :] + deg_ref[DEG_ROWS:, :] + 1.0
    return lax.transpose(lax.rsqrt(deg), (1, 0))


def _scale_rows(v, dvt):
    blocks = []
    for b in range(_NBLK):
        col = lax.slice(dvt, (0, b), (128, b + 1))
        blocks.append(v[b * 128:(b + 1) * 128, :] * col)
    col = lax.slice(dvt, (0, _NBLK), (_NTAIL, _NBLK + 1))
    blocks.append(v[_NBLK * 128:N, :] * col)
    return jnp.concatenate(blocks, axis=0)


def _mm_scale_body(x_ref, w_ref, deg_ref, hp_ref):
    dvt = _dinv_t(deg_ref)
    h = lax.dot_general(x_ref[...], w_ref[...], (((1,), (1,)), ((), ())),
                        preferred_element_type=jnp.float32)
    hp_ref[...] = _scale_rows(h, dvt)


def _mid_body(acc_ref, hp_ref, deg_ref, b_ref, g_ref, bt_ref, w_ref,
              out_ref):
    dvt = _dinv_t(deg_ref)
    acc = acc_ref[:N, :] + acc_ref[ACC_ROWS:ACC_ROWS + N, :]
    y = _scale_rows(acc + hp_ref[...], dvt) + b_ref[...]
    mean = jnp.sum(y, axis=0, keepdims=True) * (1.0 / N)
    var = jnp.sum(y * y, axis=0, keepdims=True) * (1.0 / N) - mean * mean
    z = g_ref[...] * (y - mean) * lax.rsqrt(var + EPS) + bt_ref[...]
    z = jnp.maximum(z, 0.0)
    h2 = lax.dot_general(z, w_ref[...], (((1,), (1,)), ((), ())),
                         preferred_element_type=jnp.float32)
    out_ref[...] = _scale_rows(h2, dvt)


def _final_body(acc_ref, hp_ref, deg_ref, b_ref, g_ref, bt_ref, x_ref,
                out_ref):
    dvt = _dinv_t(deg_ref)
    acc = acc_ref[:N, :] + acc_ref[ACC_ROWS:ACC_ROWS + N, :]
    y = _scale_rows(acc + hp_ref[...], dvt) + b_ref[...]
    mean = jnp.sum(y, axis=0, keepdims=True) * (1.0 / N)
    var = jnp.sum(y * y, axis=0, keepdims=True) * (1.0 / N) - mean * mean
    z = g_ref[...] * (y - mean) * lax.rsqrt(var + EPS) + bt_ref[...]
    out_ref[...] = jnp.maximum(z + x_ref[...], 0.0)


def _f32(*shapes):
    return [jax.ShapeDtypeStruct(s, jnp.float32) for s in shapes]


def kernel(x, edge_index, W1, b1, g1, bt1, W2, b2, g2, bt2):
    dst_p = edge_index[1].astype(jnp.int32)
    src_p = lax.optimization_barrier(edge_index)[0].astype(jnp.int32)

    degp = _deg_kernel(dst_p)
    deg2 = degp.reshape(NC * DEG_ROWS, 128)

    h1p = pl.pallas_call(
        _mm_scale_body,
        out_shape=_f32((N, D))[0],
    )(x, W1, deg2)

    acc1 = _edge_scatter_kernel(h1p, src_p, dst_p)

    h2p = pl.pallas_call(
        _mid_body,
        out_shape=_f32((N, D))[0],
    )(acc1.reshape(NC * ACC_ROWS, D), h1p, deg2,
      b1[None, :], g1[None, :], bt1[None, :], W2)

    acc2 = _edge_scatter_kernel(h2p, src_p, dst_p)

    out = pl.pallas_call(
        _final_body,
        out_shape=_f32((N, D))[0],
    )(acc2.reshape(NC * ACC_ROWS, D), h2p, deg2,
      b2[None, :], g2[None, :], bt2[None, :], x)
    return out
